# baseline (speedup 1.0000x reference)
import functools

import jax
import jax.numpy as jnp
import numpy as np
from jax import lax
from jax.experimental import pallas as pl
from jax.experimental.pallas import tpu as pltpu
from jax.experimental.pallas import tpu_sc as plsc

N_NODES = 10000
ROWS_PER_TILE = N_NODES // 16
E = 320000
NW = 32
CH = 128
NB = 4
NCH = E // CH
CW = NCH // NW
XTRA = NCH - NW * CW


def _seg_body(F, y_hbm, src_hbm, dst_hbm, out_hbm,
              src_v, dst_v, src_x, dst_x, rows, acc, gsems):
    cid = lax.axis_index("c")
    sid = lax.axis_index("s")
    w = sid * 2 + cid

    def zrow(i, carry):
        for j in range(F // 16):
            rows[0, i, pl.ds(j * 16, 16)] = jnp.zeros((16,), jnp.float32)
        return carry
    lax.fori_loop(0, CH, zrow, 0)
    base = sid * ROWS_PER_TILE
    full, rem = divmod(ROWS_PER_TILE, CH)
    for r in range(full):
        pltpu.sync_copy(rows.at[0], acc.at[pl.ds(base + r * CH, CH)])
    if rem:
        pltpu.sync_copy(rows.at[0, pl.ds(0, rem)],
                        acc.at[pl.ds(base + full * CH, rem)])

    pltpu.sync_copy(src_hbm.at[pl.ds(w * CW, CW)], src_v)
    pltpu.sync_copy(dst_hbm.at[pl.ds(w * CW, CW)], dst_v)

    @pl.when(w < XTRA)
    def _stage_extra():
        pltpu.sync_copy(src_hbm.at[pl.ds(NW * CW + w, 1)], src_x)
        pltpu.sync_copy(dst_hbm.at[pl.ds(NW * CW + w, 1)], dst_x)

    plsc.subcore_barrier()

    def wait_gather(b):
        pltpu.make_async_copy(y_hbm.at[src_v.at[0]], rows.at[b],
                              gsems.at[b]).wait()

    def issue_gather(j, b):
        pltpu.async_copy(y_hbm.at[src_v.at[j]], rows.at[b], gsems.at[b])

    for j in range(NB):
        issue_gather(j, j)

    main_groups = CW // NB - 1
    def outer(g, carry):
        for b in range(NB):
            j = g * NB + b
            wait_gather(b)
            pltpu.sync_copy(rows.at[b], acc.at[dst_v.at[j]], add=True)
            issue_gather(j + NB, b)
        return carry
    lax.fori_loop(0, main_groups, outer, 0)
    for j in range(main_groups * NB, CW):
        b = j % NB
        wait_gather(b)
        pltpu.sync_copy(rows.at[b], acc.at[dst_v.at[j]], add=True)
        if j + NB < CW:
            issue_gather(j + NB, b)

    @pl.when(w < XTRA)
    def _extra_chunk():
        pltpu.async_copy(y_hbm.at[src_x.at[0]], rows.at[0],
                         gsems.at[0]).wait()
        pltpu.sync_copy(rows.at[0], acc.at[dst_x.at[0]], add=True)

    plsc.subcore_barrier()
    pltpu.sync_copy(acc.at[pl.ds(base, ROWS_PER_TILE)],
                    out_hbm.at[cid, pl.ds(base, ROWS_PER_TILE), pl.ds(0, F)])


@functools.lru_cache(maxsize=None)
def _make_segsum(F):
    mesh = plsc.VectorSubcoreMesh(core_axis_name="c", subcore_axis_name="s")
    return pl.kernel(
        functools.partial(_seg_body, F),
        out_type=jax.ShapeDtypeStruct((2, N_NODES, 128), jnp.float32),
        mesh=mesh,
        scratch_types=[
            pltpu.VMEM((CW, CH), jnp.int32),
            pltpu.VMEM((CW, CH), jnp.int32),
            pltpu.VMEM((1, CH), jnp.int32),
            pltpu.VMEM((1, CH), jnp.int32),
            pltpu.VMEM((NB, CH, F), jnp.float32),
            pltpu.VMEM_SHARED((N_NODES, F), jnp.float32),
            pltpu.SemaphoreType.DMA((NB,)),
        ],
        compiler_params=pltpu.CompilerParams(use_tc_tiling_on_sc=False),
        name=f"segsum_f{F}",
    )


def _tc1_body(x_ref, wl_ref, wr_ref, b_ref, y_ref, z_ref):
    x = x_ref[...]
    y_ref[:, :64] = jnp.dot(x, wl_ref[...], preferred_element_type=jnp.float32)
    y_ref[:, 64:] = jnp.ones((N_NODES, 16), jnp.float32)
    z_ref[...] = jnp.dot(x, wr_ref[...], preferred_element_type=jnp.float32) + b_ref[...]


def _tc2_body(p_ref, z_ref, wl_ref, wr_ref, b_ref, inv_ref, y_ref, z2_ref):
    p = p_ref[0, :N_NODES, :] + p_ref[1, :N_NODES, :]
    inv = 1.0 / jnp.maximum(p[:, 64:65], 1.0)
    h = jnp.maximum(p[:, :64] * inv + z_ref[...], 0.0)
    inv_ref[...] = inv
    y_ref[...] = jnp.dot(h, wl_ref[...], preferred_element_type=jnp.float32)
    z2_ref[...] = jnp.dot(h, wr_ref[...], preferred_element_type=jnp.float32) + b_ref[...]


def _tc3_body(p_ref, z_ref, inv_ref, wl_ref, wr_ref, b_ref, y_ref, z3_ref):
    p = p_ref[0, :N_NODES, :64] + p_ref[1, :N_NODES, :64]
    h = jnp.maximum(p * inv_ref[...] + z_ref[...], 0.0)
    y_ref[...] = jnp.dot(h, wl_ref[...], preferred_element_type=jnp.float32)
    z3_ref[...] = jnp.dot(h, wr_ref[...], preferred_element_type=jnp.float32) + b_ref[...]


def _tc4_body(p_ref, z_ref, inv_ref, wc_ref, bc_ref, out_ref):
    p = p_ref[0, :N_NODES, :32] + p_ref[1, :N_NODES, :32]
    h = jnp.maximum(p * inv_ref[...] + z_ref[...], 0.0)
    out_ref[...] = jnp.dot(h, wc_ref[...], preferred_element_type=jnp.float32) + bc_ref[...]


_f32 = jnp.float32

_tc1 = pl.pallas_call(
    _tc1_body,
    out_shape=[jax.ShapeDtypeStruct((N_NODES, 80), _f32),
               jax.ShapeDtypeStruct((N_NODES, 64), _f32)])
_tc2 = pl.pallas_call(
    _tc2_body,
    out_shape=[jax.ShapeDtypeStruct((N_NODES, 1), _f32),
               jax.ShapeDtypeStruct((N_NODES, 64), _f32),
               jax.ShapeDtypeStruct((N_NODES, 64), _f32)])
_tc3 = pl.pallas_call(
    _tc3_body,
    out_shape=[jax.ShapeDtypeStruct((N_NODES, 32), _f32),
               jax.ShapeDtypeStruct((N_NODES, 32), _f32)])
_tc4 = pl.pallas_call(
    _tc4_body,
    out_shape=jax.ShapeDtypeStruct((N_NODES, 2), _f32))


def kernel(x, edge_index, Wl1, Wr1, b1, Wl2, Wr2, b2, Wl3, Wr3, b3, Wc, bc):
    src2 = edge_index[0].astype(jnp.int32).reshape(NCH, CH)
    dst2 = edge_index[1].astype(jnp.int32).reshape(NCH, CH)

    y1, z1 = _tc1(x, Wl1, Wr1, b1.reshape(1, -1))
    p1 = _make_segsum(80)(y1, src2, dst2)
    inv, y2, z2 = _tc2(p1, z1, Wl2, Wr2, b2.reshape(1, -1))
    p2 = _make_segsum(64)(y2, src2, dst2)
    y3, z3 = _tc3(p2, z2, inv, Wl3, Wr3, b3.reshape(1, -1))
    p3 = _make_segsum(32)(y3, src2, dst2)
    return _tc4(p3, z3, inv, Wc, bc.reshape(1, -1))


# speedup vs baseline: 1.0441x; 1.0441x over previous
import functools

import jax
import jax.numpy as jnp
import numpy as np
from jax import lax
from jax.experimental import pallas as pl
from jax.experimental.pallas import tpu as pltpu
from jax.experimental.pallas import tpu_sc as plsc

N_NODES = 10000
ROWS_PER_TILE = N_NODES // 16
E = 320000
NW = 32
CH = 128
NB = 4
NCH = E // CH
CW = NCH // NW
XTRA = NCH - NW * CW


def _seg_body(F, y_hbm, e_hbm, out_hbm,
              src_v, dst_v, src_x, dst_x, rows, acc, gsems):
    src_hbm = e_hbm.at[0]
    dst_hbm = e_hbm.at[1]
    cid = lax.axis_index("c")
    sid = lax.axis_index("s")
    w = sid * 2 + cid

    def zrow(i, carry):
        for j in range(F // 16):
            rows[0, i, pl.ds(j * 16, 16)] = jnp.zeros((16,), jnp.float32)
        return carry
    lax.fori_loop(0, CH, zrow, 0)
    base = sid * ROWS_PER_TILE
    full, rem = divmod(ROWS_PER_TILE, CH)
    for r in range(full):
        pltpu.sync_copy(rows.at[0], acc.at[pl.ds(base + r * CH, CH)])
    if rem:
        pltpu.sync_copy(rows.at[0, pl.ds(0, rem)],
                        acc.at[pl.ds(base + full * CH, rem)])

    pltpu.sync_copy(src_hbm.at[pl.ds(w * CW, CW)], src_v)
    pltpu.sync_copy(dst_hbm.at[pl.ds(w * CW, CW)], dst_v)

    @pl.when(w < XTRA)
    def _stage_extra():
        pltpu.sync_copy(src_hbm.at[pl.ds(NW * CW + w, 1)], src_x)
        pltpu.sync_copy(dst_hbm.at[pl.ds(NW * CW + w, 1)], dst_x)

    plsc.subcore_barrier()

    def wait_gather(b):
        pltpu.make_async_copy(y_hbm.at[src_v.at[0]], rows.at[b],
                              gsems.at[b]).wait()

    def issue_gather(j, b):
        pltpu.async_copy(y_hbm.at[src_v.at[j]], rows.at[b], gsems.at[b])

    for j in range(NB):
        issue_gather(j, j)

    main_groups = CW // NB - 1
    def outer(g, carry):
        for b in range(NB):
            j = g * NB + b
            wait_gather(b)
            pltpu.sync_copy(rows.at[b], acc.at[dst_v.at[j]], add=True)
            issue_gather(j + NB, b)
        return carry
    lax.fori_loop(0, main_groups, outer, 0)
    for j in range(main_groups * NB, CW):
        b = j % NB
        wait_gather(b)
        pltpu.sync_copy(rows.at[b], acc.at[dst_v.at[j]], add=True)
        if j + NB < CW:
            issue_gather(j + NB, b)

    @pl.when(w < XTRA)
    def _extra_chunk():
        pltpu.async_copy(y_hbm.at[src_x.at[0]], rows.at[0],
                         gsems.at[0]).wait()
        pltpu.sync_copy(rows.at[0], acc.at[dst_x.at[0]], add=True)

    plsc.subcore_barrier()
    pltpu.sync_copy(acc.at[pl.ds(base, ROWS_PER_TILE)],
                    out_hbm.at[cid, pl.ds(base, ROWS_PER_TILE), pl.ds(0, F)])


@functools.lru_cache(maxsize=None)
def _make_segsum(F):
    mesh = plsc.VectorSubcoreMesh(core_axis_name="c", subcore_axis_name="s")
    return pl.kernel(
        functools.partial(_seg_body, F),
        out_type=jax.ShapeDtypeStruct((2, N_NODES, 128), jnp.float32),
        mesh=mesh,
        scratch_types=[
            pltpu.VMEM((CW, CH), jnp.int32),
            pltpu.VMEM((CW, CH), jnp.int32),
            pltpu.VMEM((1, CH), jnp.int32),
            pltpu.VMEM((1, CH), jnp.int32),
            pltpu.VMEM((NB, CH, F), jnp.float32),
            pltpu.VMEM_SHARED((N_NODES, F), jnp.float32),
            pltpu.SemaphoreType.DMA((NB,)),
        ],
        compiler_params=pltpu.CompilerParams(use_tc_tiling_on_sc=False),
        name=f"segsum_f{F}",
    )


def _tc1_body(x_ref, wl_ref, wr_ref, b_ref, y_ref, z_ref):
    x = x_ref[...]
    y_ref[:, :64] = jnp.dot(x, wl_ref[...], preferred_element_type=jnp.float32)
    y_ref[:, 64:] = jnp.ones((N_NODES, 16), jnp.float32)
    z_ref[...] = jnp.dot(x, wr_ref[...], preferred_element_type=jnp.float32) + b_ref[...]


def _tc2_body(p_ref, z_ref, wl_ref, wr_ref, b_ref, inv_ref, y_ref, z2_ref):
    p = p_ref[0, :N_NODES, :] + p_ref[1, :N_NODES, :]
    inv = 1.0 / jnp.maximum(p[:, 64:65], 1.0)
    h = jnp.maximum(p[:, :64] * inv + z_ref[...], 0.0)
    inv_ref[...] = inv
    y_ref[...] = jnp.dot(h, wl_ref[...], preferred_element_type=jnp.float32)
    z2_ref[...] = jnp.dot(h, wr_ref[...], preferred_element_type=jnp.float32) + b_ref[...]


def _tc3_body(p_ref, z_ref, inv_ref, wl_ref, wr_ref, b_ref, y_ref, z3_ref):
    p = p_ref[0, :N_NODES, :64] + p_ref[1, :N_NODES, :64]
    h = jnp.maximum(p * inv_ref[...] + z_ref[...], 0.0)
    y_ref[...] = jnp.dot(h, wl_ref[...], preferred_element_type=jnp.float32)
    z3_ref[...] = jnp.dot(h, wr_ref[...], preferred_element_type=jnp.float32) + b_ref[...]


def _tc4_body(p_ref, z_ref, inv_ref, wc_ref, bc_ref, out_ref):
    p = p_ref[0, :N_NODES, :32] + p_ref[1, :N_NODES, :32]
    h = jnp.maximum(p * inv_ref[...] + z_ref[...], 0.0)
    out_ref[...] = jnp.dot(h, wc_ref[...], preferred_element_type=jnp.float32) + bc_ref[...]


_f32 = jnp.float32

_tc1 = pl.pallas_call(
    _tc1_body,
    out_shape=[jax.ShapeDtypeStruct((N_NODES, 80), _f32),
               jax.ShapeDtypeStruct((N_NODES, 64), _f32)])
_tc2 = pl.pallas_call(
    _tc2_body,
    out_shape=[jax.ShapeDtypeStruct((N_NODES, 1), _f32),
               jax.ShapeDtypeStruct((N_NODES, 64), _f32),
               jax.ShapeDtypeStruct((N_NODES, 64), _f32)])
_tc3 = pl.pallas_call(
    _tc3_body,
    out_shape=[jax.ShapeDtypeStruct((N_NODES, 32), _f32),
               jax.ShapeDtypeStruct((N_NODES, 32), _f32)])
_tc4 = pl.pallas_call(
    _tc4_body,
    out_shape=jax.ShapeDtypeStruct((N_NODES, 2), _f32))


def kernel(x, edge_index, Wl1, Wr1, b1, Wl2, Wr2, b2, Wl3, Wr3, b3, Wc, bc):
    e3 = edge_index.astype(jnp.int32).reshape(2, NCH, CH)

    y1, z1 = _tc1(x, Wl1, Wr1, b1.reshape(1, -1))
    p1 = _make_segsum(80)(y1, e3)
    inv, y2, z2 = _tc2(p1, z1, Wl2, Wr2, b2.reshape(1, -1))
    p2 = _make_segsum(64)(y2, e3)
    y3, z3 = _tc3(p2, z2, inv, Wl3, Wr3, b3.reshape(1, -1))
    p3 = _make_segsum(32)(y3, e3)
    return _tc4(p3, z3, inv, Wc, bc.reshape(1, -1))
